# baseline (speedup 1.0000x reference)
import functools

import numpy as np
import jax
import jax.numpy as jnp
from jax import lax
from jax.experimental import pallas as pl
from jax.experimental.pallas import tpu as pltpu
from jax.experimental.pallas import tpu_sc as plsc

_B, _C, _D = 4096, 14, 1024
_ROWS = _B * _C
_NW = 32
_RPW = _ROWS // _NW
_CHUNK = 112
_NCH = _RPW // _CHUNK


@functools.lru_cache(maxsize=None)
def _row_index_host() -> np.ndarray:
    with jax.ensure_compile_time_eval():
        perm = np.asarray(jax.random.permutation(jax.random.key(42), 6))
    src = np.arange(_C)
    src[0:6] = perm
    src[7:13] = 7 + perm
    r = np.arange(_ROWS)
    idx = (r // _C) * _C + src[r % _C]
    return idx.astype(np.int32).reshape(_NW * _NCH, _CHUNK)


_mesh = plsc.VectorSubcoreMesh(core_axis_name="c", subcore_axis_name="s")


@functools.partial(
    pl.kernel,
    mesh=_mesh,
    out_type=jax.ShapeDtypeStruct((_ROWS, _D), jnp.float32),
    scratch_types=[
        pltpu.VMEM((_NCH, _CHUNK), jnp.int32),
        pltpu.VMEM((_CHUNK, _D), jnp.float32),
        pltpu.SemaphoreType.DMA,
    ],
)
def _shuffle(in_hbm, idx_hbm, out_hbm, idx_v, buf, sem):
    wid = lax.axis_index("s") * 2 + lax.axis_index("c")
    pltpu.sync_copy(idx_hbm.at[pl.ds(wid * _NCH, _NCH)], idx_v)
    base = wid * _RPW
    for j in range(_NCH):
        pltpu.async_copy(in_hbm.at[idx_v.at[j]], buf, sem).wait()
        pltpu.sync_copy(buf, out_hbm.at[pl.ds(base + j * _CHUNK, _CHUNK)])


def kernel(chromatogram_batch):
    x = chromatogram_batch.reshape(_ROWS, _D)
    idx = jnp.asarray(_row_index_host())
    out = _shuffle(x, idx)
    return out.reshape(_B, _C, _D)


# speedup vs baseline: 1.0063x; 1.0063x over previous
import functools

import numpy as np
import jax
import jax.numpy as jnp
from jax import lax
from jax.experimental import pallas as pl
from jax.experimental.pallas import tpu as pltpu
from jax.experimental.pallas import tpu_sc as plsc

_B, _C, _D = 4096, 14, 1024
_ROWS = _B * _C
_NW = 32
_RPW = _ROWS // _NW
_CHUNK = 56
_NCH = _RPW // _CHUNK


@functools.lru_cache(maxsize=None)
def _row_index_host() -> np.ndarray:
    with jax.ensure_compile_time_eval():
        perm = np.asarray(jax.random.permutation(jax.random.key(42), 6))
    src = np.arange(_C)
    src[0:6] = perm
    src[7:13] = 7 + perm
    r = np.arange(_ROWS)
    idx = (r // _C) * _C + src[r % _C]
    return idx.astype(np.int32).reshape(_NW * _NCH, _CHUNK)


_mesh = plsc.VectorSubcoreMesh(core_axis_name="c", subcore_axis_name="s")


@functools.partial(
    pl.kernel,
    mesh=_mesh,
    out_type=jax.ShapeDtypeStruct((_ROWS, _D), jnp.float32),
    scratch_types=[
        pltpu.VMEM((_NCH, _CHUNK), jnp.int32),
        pltpu.VMEM((_CHUNK, _D), jnp.float32),
        pltpu.VMEM((_CHUNK, _D), jnp.float32),
        pltpu.SemaphoreType.DMA,
        pltpu.SemaphoreType.DMA,
        pltpu.SemaphoreType.DMA,
        pltpu.SemaphoreType.DMA,
    ],
)
def _shuffle(in_hbm, idx_hbm, out_hbm, idx_v, buf0, buf1, gsem0, gsem1,
             ssem0, ssem1):
    wid = lax.axis_index("s") * 2 + lax.axis_index("c")
    pltpu.sync_copy(idx_hbm.at[pl.ds(wid * _NCH, _NCH)], idx_v)
    base = wid * _RPW

    def gather(j, buf, sem):
        return pltpu.async_copy(in_hbm.at[idx_v.at[j]], buf, sem)

    def store(j, buf, sem):
        return pltpu.async_copy(
            buf, out_hbm.at[pl.ds(base + j * _CHUNK, _CHUNK)], sem)

    gather(0, buf0, gsem0)
    gather(1, buf1, gsem1)

    @pl.loop(0, _NCH - 2, step=2)
    def _steady(jj):
        pltpu.make_async_copy(in_hbm.at[idx_v.at[jj]], buf0, gsem0).wait()
        s0 = store(jj, buf0, ssem0)
        pltpu.make_async_copy(in_hbm.at[idx_v.at[jj + 1]], buf1, gsem1).wait()
        s1 = store(jj + 1, buf1, ssem1)
        s0.wait()
        gather(jj + 2, buf0, gsem0)
        s1.wait()
        gather(jj + 3, buf1, gsem1)

    pltpu.make_async_copy(in_hbm.at[idx_v.at[_NCH - 2]], buf0, gsem0).wait()
    s0 = store(_NCH - 2, buf0, ssem0)
    pltpu.make_async_copy(in_hbm.at[idx_v.at[_NCH - 1]], buf1, gsem1).wait()
    s1 = store(_NCH - 1, buf1, ssem1)
    s0.wait()
    s1.wait()


def kernel(chromatogram_batch):
    x = chromatogram_batch.reshape(_ROWS, _D)
    idx = jnp.asarray(_row_index_host())
    out = _shuffle(x, idx)
    return out.reshape(_B, _C, _D)
